# baseline (speedup 1.0000x reference)
import functools

import jax
import jax.numpy as jnp
from jax import lax
from jax.experimental import pallas as pl
from jax.experimental.pallas import tpu as pltpu
from jax.experimental.pallas import tpu_sc as plsc

N = 10000
E = 320000
D = 128
NC = 2
NS = 16
NW = NC * NS
EPT = E // NW
CHUNK = 128
NCHUNK = 81
EPT_PAD = NCHUNK * CHUNK
ACC_ROWS = 10112
TRASH = N + 100
ZROWS_PER_TILE = ACC_ROWS // NS
DEG_W = 128
ROWBLK = 2000


def _sc_mesh():
    return plsc.VectorSubcoreMesh(core_axis_name="c", subcore_axis_name="s")


def _zero_acc(zeros_hbm, buf, acc, s):
    pltpu.sync_copy(zeros_hbm, buf)
    base = s * ZROWS_PER_TILE
    nfull = ZROWS_PER_TILE // CHUNK
    for z in range(nfull):
        pltpu.sync_copy(buf, acc.at[pl.ds(base + z * CHUNK, CHUNK)])
    rem = ZROWS_PER_TILE - nfull * CHUNK
    if rem:
        pltpu.sync_copy(buf.at[pl.ds(0, rem)], acc.at[pl.ds(base + nfull * CHUNK, rem)])


def _deg_body(rows_hbm, zeros_hbm, ones_hbm, out_hbm,
              ridx0, ridx1, ridx2, buf, acc, sr0, sr1, sr2):
    c = lax.axis_index("c")
    s = lax.axis_index("s")
    wid = c * NS + s
    _zero_acc(zeros_hbm, buf, acc, s)
    pltpu.sync_copy(ones_hbm, buf)
    plsc.subcore_barrier()

    ridx = (ridx0, ridx1, ridx2)
    srs = (sr0, sr1, sr2)
    for x in range(3):
        pltpu.async_copy(rows_hbm.at[wid, x], ridx[x], srs[x])

    def body(k, carry):
        for off in range(3):
            j = k * 3 + off
            pltpu.make_async_copy(rows_hbm.at[wid, j], ridx[off], srs[off]).wait()
            pltpu.sync_copy(buf, acc.at[ridx[off]], add=True)

            @pl.when(j + 3 < NCHUNK)
            def _i3():
                pltpu.async_copy(rows_hbm.at[wid, j + 3], ridx[off], srs[off])

        return carry

    lax.fori_loop(0, NCHUNK // 3, body, 0)
    plsc.subcore_barrier()
    pltpu.sync_copy(acc.at[pl.ds(s * ZROWS_PER_TILE, ZROWS_PER_TILE)],
                    out_hbm.at[c, pl.ds(s * ZROWS_PER_TILE, ZROWS_PER_TILE)])


def _hop_body(g_hbm, ci_hbm, zeros_hbm, out_hbm,
              idx0, idx1, idx2, buf0, buf1, buf2, acc,
              si0, si1, si2, sg0, sg1, sg2):
    c = lax.axis_index("c")
    s = lax.axis_index("s")
    wid = c * NS + s
    _zero_acc(zeros_hbm, buf0, acc, s)
    plsc.subcore_barrier()

    idx = (idx0, idx1, idx2)
    buf = (buf0, buf1, buf2)
    sis = (si0, si1, si2)
    sgs = (sg0, sg1, sg2)

    for x in range(3):
        pltpu.async_copy(ci_hbm.at[wid, x], idx[x], sis[x])
    for x in range(2):
        pltpu.make_async_copy(ci_hbm.at[wid, x], idx[x], sis[x]).wait()
        pltpu.async_copy(g_hbm.at[idx[x].at[0]], buf[x], sgs[x])

    def body(k, carry):
        for off in range(3):
            j = k * 3 + off
            x2 = (off + 2) % 3

            @pl.when(j + 2 < NCHUNK)
            def _g2():
                pltpu.make_async_copy(ci_hbm.at[wid, j + 2], idx[x2], sis[x2]).wait()
                pltpu.async_copy(g_hbm.at[idx[x2].at[0]], buf[x2], sgs[x2])

            pltpu.make_async_copy(g_hbm.at[idx[off].at[0]], buf[off], sgs[off]).wait()
            pltpu.sync_copy(buf[off], acc.at[idx[off].at[1]], add=True)

            @pl.when(j + 3 < NCHUNK)
            def _i3():
                pltpu.async_copy(ci_hbm.at[wid, j + 3], idx[off], sis[off])

        return carry

    lax.fori_loop(0, NCHUNK // 3, body, 0)
    plsc.subcore_barrier()
    pltpu.sync_copy(acc.at[pl.ds(s * ZROWS_PER_TILE, ZROWS_PER_TILE)],
                    out_hbm.at[c, pl.ds(s * ZROWS_PER_TILE, ZROWS_PER_TILE)])


def _run_deg(rows_pad):
    fn = pl.kernel(
        _deg_body,
        out_type=jax.ShapeDtypeStruct((NC, ACC_ROWS, DEG_W), jnp.float32),
        mesh=_sc_mesh(),
        scratch_types=[
            pltpu.VMEM((CHUNK,), jnp.int32),
            pltpu.VMEM((CHUNK,), jnp.int32),
            pltpu.VMEM((CHUNK,), jnp.int32),
            pltpu.VMEM((CHUNK, DEG_W), jnp.float32),
            pltpu.VMEM_SHARED((ACC_ROWS, DEG_W), jnp.float32),
            pltpu.SemaphoreType.DMA,
            pltpu.SemaphoreType.DMA,
            pltpu.SemaphoreType.DMA,
        ],
    )
    zeros = jnp.zeros((CHUNK, DEG_W), jnp.float32)
    ones = jnp.ones((CHUNK, DEG_W), jnp.float32)
    return fn(rows_pad, zeros, ones)


def _run_hop(g, ci):
    fn = pl.kernel(
        _hop_body,
        out_type=jax.ShapeDtypeStruct((NC, ACC_ROWS, D), jnp.float32),
        mesh=_sc_mesh(),
        scratch_types=[
            pltpu.VMEM((2, CHUNK), jnp.int32),
            pltpu.VMEM((2, CHUNK), jnp.int32),
            pltpu.VMEM((2, CHUNK), jnp.int32),
            pltpu.VMEM((CHUNK, D), jnp.float32),
            pltpu.VMEM((CHUNK, D), jnp.float32),
            pltpu.VMEM((CHUNK, D), jnp.float32),
            pltpu.VMEM_SHARED((ACC_ROWS, D), jnp.float32),
            pltpu.SemaphoreType.DMA,
            pltpu.SemaphoreType.DMA,
            pltpu.SemaphoreType.DMA,
            pltpu.SemaphoreType.DMA,
            pltpu.SemaphoreType.DMA,
            pltpu.SemaphoreType.DMA,
        ],
    )
    zeros = jnp.zeros((CHUNK, D), jnp.float32)
    return fn(g, ci, zeros)


def _mm_body(x_ref, w_ref, y_ref):
    y_ref[...] = lax.dot_general(x_ref[...], w_ref[...],
                                 (((1,), (1,)), ((), ())),
                                 preferred_element_type=jnp.float32)


def _run_mm(x, W):
    grid = N // ROWBLK
    return pl.pallas_call(
        _mm_body,
        grid=(grid,),
        in_specs=[
            pl.BlockSpec((ROWBLK, D), lambda i: (i, 0)),
            pl.BlockSpec((D, D), lambda i: (0, 0)),
        ],
        out_specs=pl.BlockSpec((ROWBLK, D), lambda i: (i, 0)),
        out_shape=jax.ShapeDtypeStruct((N, D), jnp.float32),
    )(x, W)


def _deg_from_parts(dp_ref):
    return dp_ref[0, :, 0:1] + dp_ref[1, :, 0:1] + 1.0


def _scale_body(dp_ref, y_ref, g_ref):
    g_ref[...] = y_ref[...] * lax.rsqrt(_deg_from_parts(dp_ref))


def _run_scale(dp, y):
    grid = N // ROWBLK
    return pl.pallas_call(
        _scale_body,
        grid=(grid,),
        in_specs=[
            pl.BlockSpec((NC, ROWBLK, DEG_W), lambda i: (0, i, 0)),
            pl.BlockSpec((ROWBLK, D), lambda i: (i, 0)),
        ],
        out_specs=pl.BlockSpec((ROWBLK, D), lambda i: (i, 0)),
        out_shape=jax.ShapeDtypeStruct((N, D), jnp.float32),
    )(dp, y)


def _mid_body(dp_ref, p_ref, g0_ref, g1_ref):
    g1_ref[...] = (p_ref[0] + p_ref[1] + g0_ref[...]) / _deg_from_parts(dp_ref)


def _run_mid(dp, p, g0):
    grid = N // ROWBLK
    return pl.pallas_call(
        _mid_body,
        grid=(grid,),
        in_specs=[
            pl.BlockSpec((NC, ROWBLK, DEG_W), lambda i: (0, i, 0)),
            pl.BlockSpec((NC, ROWBLK, D), lambda i: (0, i, 0)),
            pl.BlockSpec((ROWBLK, D), lambda i: (i, 0)),
        ],
        out_specs=pl.BlockSpec((ROWBLK, D), lambda i: (i, 0)),
        out_shape=jax.ShapeDtypeStruct((N, D), jnp.float32),
    )(dp, p, g0)


def _final_body(dp_ref, q_ref, g1_ref, b_ref, o_ref):
    z = ((q_ref[0] + q_ref[1] + g1_ref[...]) * lax.rsqrt(_deg_from_parts(dp_ref))
         + b_ref[...])
    m = jnp.max(z, axis=1, keepdims=True)
    lse = jnp.log(jnp.sum(jnp.exp(z - m), axis=1, keepdims=True)) + m
    o_ref[...] = z - lse


def _run_final(dp, q, g1, b):
    grid = N // ROWBLK
    return pl.pallas_call(
        _final_body,
        grid=(grid,),
        in_specs=[
            pl.BlockSpec((NC, ROWBLK, DEG_W), lambda i: (0, i, 0)),
            pl.BlockSpec((NC, ROWBLK, D), lambda i: (0, i, 0)),
            pl.BlockSpec((ROWBLK, D), lambda i: (i, 0)),
            pl.BlockSpec((1, D), lambda i: (0, 0)),
        ],
        out_specs=pl.BlockSpec((ROWBLK, D), lambda i: (i, 0)),
        out_shape=jax.ShapeDtypeStruct((N, D), jnp.float32),
    )(dp, q, g1, b)


def kernel(x, edge_index, W, b):
    row = edge_index[0].reshape(NW, EPT)
    col = edge_index[1].reshape(NW, EPT)
    pad = EPT_PAD - EPT
    rows_pad = jnp.pad(row, ((0, 0), (0, pad)), constant_values=TRASH)
    cols_pad = jnp.pad(col, ((0, 0), (0, pad)), constant_values=0)
    rows_pad = rows_pad.reshape(NW, NCHUNK, CHUNK)
    cols_pad = cols_pad.reshape(NW, NCHUNK, CHUNK)
    ci = jnp.stack([cols_pad, rows_pad], axis=2)

    dp = _run_deg(rows_pad)
    y = _run_mm(x, W)
    g0 = _run_scale(dp, y)
    p = _run_hop(g0, ci)
    g1 = _run_mid(dp, p, g0)
    q = _run_hop(g1, ci)
    return _run_final(dp, q, g1, b.reshape(1, D))


# speedup vs baseline: 1.2666x; 1.2666x over previous
import functools

import jax
import jax.numpy as jnp
from jax import lax
from jax.experimental import pallas as pl
from jax.experimental.pallas import tpu as pltpu
from jax.experimental.pallas import tpu_sc as plsc

N = 10000
E = 320000
D = 128
NC = 2
NS = 16
NW = NC * NS
EPT = E // NW
CHUNK = 128
NCHUNK = 80
EPT_PAD = NCHUNK * CHUNK
ACC_ROWS = 10240
TRASH = N + 100
ZROWS_PER_TILE = ACC_ROWS // NS
DEG_W = 128
ROWBLK = 2000


def _sc_mesh():
    return plsc.VectorSubcoreMesh(core_axis_name="c", subcore_axis_name="s")


def _deg_body(rows_hbm, zeros_hbm, ones_hbm, out_hbm,
              ridx0, ridx1, buf, acc, sr0, sr1):
    c = lax.axis_index("c")
    s = lax.axis_index("s")
    wid = c * NS + s
    pltpu.sync_copy(zeros_hbm, buf)
    for z in range(ZROWS_PER_TILE // CHUNK):
        pltpu.sync_copy(buf, acc.at[pl.ds(s * ZROWS_PER_TILE + z * CHUNK, CHUNK)])
    pltpu.sync_copy(ones_hbm, buf)
    plsc.subcore_barrier()
    pltpu.async_copy(rows_hbm.at[wid, 0], ridx0, sr0)
    pltpu.async_copy(rows_hbm.at[wid, 1], ridx1, sr1)

    def body(jj, carry):
        j = jj * 2
        pltpu.make_async_copy(rows_hbm.at[wid, j], ridx0, sr0).wait()
        pltpu.sync_copy(buf, acc.at[ridx0], add=True)

        @pl.when(j + 2 < NCHUNK)
        def _pf0():
            pltpu.async_copy(rows_hbm.at[wid, j + 2], ridx0, sr0)

        pltpu.make_async_copy(rows_hbm.at[wid, j + 1], ridx1, sr1).wait()
        pltpu.sync_copy(buf, acc.at[ridx1], add=True)

        @pl.when(j + 3 < NCHUNK)
        def _pf1():
            pltpu.async_copy(rows_hbm.at[wid, j + 3], ridx1, sr1)

        return carry

    lax.fori_loop(0, NCHUNK // 2, body, 0)
    plsc.subcore_barrier()
    pltpu.sync_copy(acc.at[pl.ds(s * ZROWS_PER_TILE, ZROWS_PER_TILE)],
                    out_hbm.at[c, pl.ds(s * ZROWS_PER_TILE, ZROWS_PER_TILE)])


def _hop_body(g_hbm, rows_hbm, cols_hbm, zeros_hbm, out_hbm,
              cols_v, ridx0, ridx1, buf0, buf1, acc,
              sg0, sg1, sr0, sr1, ss0, ss1):
    c = lax.axis_index("c")
    s = lax.axis_index("s")
    wid = c * NS + s
    pltpu.sync_copy(zeros_hbm, buf0)
    for z in range(ZROWS_PER_TILE // CHUNK):
        pltpu.sync_copy(buf0, acc.at[pl.ds(s * ZROWS_PER_TILE + z * CHUNK, CHUNK)])
    pltpu.sync_copy(cols_hbm.at[wid], cols_v)
    plsc.subcore_barrier()

    pltpu.async_copy(g_hbm.at[cols_v.at[0]], buf0, sg0)
    pltpu.async_copy(g_hbm.at[cols_v.at[1]], buf1, sg1)
    pltpu.async_copy(rows_hbm.at[wid, 0], ridx0, sr0)
    pltpu.async_copy(rows_hbm.at[wid, 1], ridx1, sr1)

    def body(jj, carry):
        j = jj * 2
        pltpu.make_async_copy(g_hbm.at[cols_v.at[j]], buf0, sg0).wait()
        pltpu.make_async_copy(rows_hbm.at[wid, j], ridx0, sr0).wait()
        pltpu.async_copy(buf0, acc.at[ridx0], ss0, add=True)

        pltpu.make_async_copy(g_hbm.at[cols_v.at[j + 1]], buf1, sg1).wait()
        pltpu.make_async_copy(rows_hbm.at[wid, j + 1], ridx1, sr1).wait()
        pltpu.async_copy(buf1, acc.at[ridx1], ss1, add=True)

        @pl.when(j + 2 < NCHUNK)
        def _pf0():
            pltpu.make_async_copy(buf0, acc.at[ridx0], ss0).wait()
            pltpu.async_copy(g_hbm.at[cols_v.at[j + 2]], buf0, sg0)
            pltpu.async_copy(rows_hbm.at[wid, j + 2], ridx0, sr0)

        @pl.when(j + 3 < NCHUNK)
        def _pf1():
            pltpu.make_async_copy(buf1, acc.at[ridx1], ss1).wait()
            pltpu.async_copy(g_hbm.at[cols_v.at[j + 3]], buf1, sg1)
            pltpu.async_copy(rows_hbm.at[wid, j + 3], ridx1, sr1)

        return carry

    lax.fori_loop(0, NCHUNK // 2, body, 0)
    pltpu.make_async_copy(buf0, acc.at[ridx0], ss0).wait()
    pltpu.make_async_copy(buf1, acc.at[ridx1], ss1).wait()
    plsc.subcore_barrier()
    pltpu.sync_copy(acc.at[pl.ds(s * ZROWS_PER_TILE, ZROWS_PER_TILE)],
                    out_hbm.at[c, pl.ds(s * ZROWS_PER_TILE, ZROWS_PER_TILE)])


def _run_deg(rows_pad):
    fn = pl.kernel(
        _deg_body,
        out_type=jax.ShapeDtypeStruct((NC, ACC_ROWS, DEG_W), jnp.float32),
        mesh=_sc_mesh(),
        scratch_types=[
            pltpu.VMEM((CHUNK,), jnp.int32),
            pltpu.VMEM((CHUNK,), jnp.int32),
            pltpu.VMEM((CHUNK, DEG_W), jnp.float32),
            pltpu.VMEM_SHARED((ACC_ROWS, DEG_W), jnp.float32),
            pltpu.SemaphoreType.DMA,
            pltpu.SemaphoreType.DMA,
        ],
    )
    zeros = jnp.zeros((CHUNK, DEG_W), jnp.float32)
    ones = jnp.ones((CHUNK, DEG_W), jnp.float32)
    return fn(rows_pad, zeros, ones)


def _run_hop(g, rows_pad, cols_pad):
    fn = pl.kernel(
        _hop_body,
        out_type=jax.ShapeDtypeStruct((NC, ACC_ROWS, D), jnp.float32),
        mesh=_sc_mesh(),
        scratch_types=[
            pltpu.VMEM((NCHUNK, CHUNK), jnp.int32),
            pltpu.VMEM((CHUNK,), jnp.int32),
            pltpu.VMEM((CHUNK,), jnp.int32),
            pltpu.VMEM((CHUNK, D), jnp.float32),
            pltpu.VMEM((CHUNK, D), jnp.float32),
            pltpu.VMEM_SHARED((ACC_ROWS, D), jnp.float32),
            pltpu.SemaphoreType.DMA,
            pltpu.SemaphoreType.DMA,
            pltpu.SemaphoreType.DMA,
            pltpu.SemaphoreType.DMA,
            pltpu.SemaphoreType.DMA,
            pltpu.SemaphoreType.DMA,
        ],
    )
    zeros = jnp.zeros((CHUNK, D), jnp.float32)
    return fn(g, rows_pad, cols_pad, zeros)


def _mm_body(x_ref, w_ref, y_ref):
    y_ref[...] = lax.dot_general(x_ref[...], w_ref[...],
                                 (((1,), (1,)), ((), ())),
                                 preferred_element_type=jnp.float32)


def _run_mm(x, W):
    grid = N // ROWBLK
    return pl.pallas_call(
        _mm_body,
        grid=(grid,),
        in_specs=[
            pl.BlockSpec((ROWBLK, D), lambda i: (i, 0)),
            pl.BlockSpec((D, D), lambda i: (0, 0)),
        ],
        out_specs=pl.BlockSpec((ROWBLK, D), lambda i: (i, 0)),
        out_shape=jax.ShapeDtypeStruct((N, D), jnp.float32),
    )(x, W)


def _deg_from_parts(dp_ref):
    return dp_ref[0, :, 0:1] + dp_ref[1, :, 0:1] + 1.0


def _scale_body(dp_ref, y_ref, g_ref):
    g_ref[...] = y_ref[...] * lax.rsqrt(_deg_from_parts(dp_ref))


def _run_scale(dp, y):
    grid = N // ROWBLK
    return pl.pallas_call(
        _scale_body,
        grid=(grid,),
        in_specs=[
            pl.BlockSpec((NC, ROWBLK, DEG_W), lambda i: (0, i, 0)),
            pl.BlockSpec((ROWBLK, D), lambda i: (i, 0)),
        ],
        out_specs=pl.BlockSpec((ROWBLK, D), lambda i: (i, 0)),
        out_shape=jax.ShapeDtypeStruct((N, D), jnp.float32),
    )(dp, y)


def _mid_body(dp_ref, p_ref, g0_ref, g1_ref):
    g1_ref[...] = (p_ref[0] + p_ref[1] + g0_ref[...]) / _deg_from_parts(dp_ref)


def _run_mid(dp, p, g0):
    grid = N // ROWBLK
    return pl.pallas_call(
        _mid_body,
        grid=(grid,),
        in_specs=[
            pl.BlockSpec((NC, ROWBLK, DEG_W), lambda i: (0, i, 0)),
            pl.BlockSpec((NC, ROWBLK, D), lambda i: (0, i, 0)),
            pl.BlockSpec((ROWBLK, D), lambda i: (i, 0)),
        ],
        out_specs=pl.BlockSpec((ROWBLK, D), lambda i: (i, 0)),
        out_shape=jax.ShapeDtypeStruct((N, D), jnp.float32),
    )(dp, p, g0)


def _final_body(dp_ref, q_ref, g1_ref, b_ref, o_ref):
    z = ((q_ref[0] + q_ref[1] + g1_ref[...]) * lax.rsqrt(_deg_from_parts(dp_ref))
         + b_ref[...])
    m = jnp.max(z, axis=1, keepdims=True)
    lse = jnp.log(jnp.sum(jnp.exp(z - m), axis=1, keepdims=True)) + m
    o_ref[...] = z - lse


def _run_final(dp, q, g1, b):
    grid = N // ROWBLK
    return pl.pallas_call(
        _final_body,
        grid=(grid,),
        in_specs=[
            pl.BlockSpec((NC, ROWBLK, DEG_W), lambda i: (0, i, 0)),
            pl.BlockSpec((NC, ROWBLK, D), lambda i: (0, i, 0)),
            pl.BlockSpec((ROWBLK, D), lambda i: (i, 0)),
            pl.BlockSpec((1, D), lambda i: (0, 0)),
        ],
        out_specs=pl.BlockSpec((ROWBLK, D), lambda i: (i, 0)),
        out_shape=jax.ShapeDtypeStruct((N, D), jnp.float32),
    )(dp, q, g1, b)


def kernel(x, edge_index, W, b):
    row = edge_index[0].reshape(NW, EPT)
    col = edge_index[1].reshape(NW, EPT)
    pad = EPT_PAD - EPT
    rows_pad = jnp.pad(row, ((0, 0), (0, pad)), constant_values=TRASH)
    cols_pad = jnp.pad(col, ((0, 0), (0, pad)), constant_values=0)
    rows_pad = rows_pad.reshape(NW, NCHUNK, CHUNK)
    cols_pad = cols_pad.reshape(NW, NCHUNK, CHUNK)

    dp = _run_deg(rows_pad)
    y = _run_mm(x, W)
    g0 = _run_scale(dp, y)
    p = _run_hop(g0, rows_pad, cols_pad)
    g1 = _run_mid(dp, p, g0)
    q = _run_hop(g1, rows_pad, cols_pad)
    return _run_final(dp, q, g1, b.reshape(1, D))


# speedup vs baseline: 1.3632x; 1.0763x over previous
import functools

import jax
import jax.numpy as jnp
from jax import lax
from jax.experimental import pallas as pl
from jax.experimental.pallas import tpu as pltpu
from jax.experimental.pallas import tpu_sc as plsc

N = 10000
E = 320000
D = 128
NC = 2
NS = 16
NW = NC * NS
EPT = E // NW
CHUNK = 128
NCHUNK = 80
EPT_PAD = NCHUNK * CHUNK
ACC_ROWS = 10240
TRASH = N + 100
ZROWS_PER_TILE = ACC_ROWS // NS
DEG_W = 128
ROWBLK = 2000


def _sc_mesh():
    return plsc.VectorSubcoreMesh(core_axis_name="c", subcore_axis_name="s")


def _deg_body(rows_hbm, zeros_hbm, ones_hbm, out_hbm,
              ridx0, ridx1, buf, acc, sr0, sr1):
    c = lax.axis_index("c")
    s = lax.axis_index("s")
    wid = c * NS + s
    pltpu.sync_copy(zeros_hbm, buf)
    for z in range(ZROWS_PER_TILE // CHUNK):
        pltpu.sync_copy(buf, acc.at[pl.ds(s * ZROWS_PER_TILE + z * CHUNK, CHUNK)])
    pltpu.sync_copy(ones_hbm, buf)
    plsc.subcore_barrier()
    pltpu.async_copy(rows_hbm.at[wid, 0], ridx0, sr0)
    pltpu.async_copy(rows_hbm.at[wid, 1], ridx1, sr1)

    def body(jj, carry):
        j = jj * 2
        pltpu.make_async_copy(rows_hbm.at[wid, j], ridx0, sr0).wait()
        pltpu.sync_copy(buf, acc.at[ridx0], add=True)

        @pl.when(j + 2 < NCHUNK)
        def _pf0():
            pltpu.async_copy(rows_hbm.at[wid, j + 2], ridx0, sr0)

        pltpu.make_async_copy(rows_hbm.at[wid, j + 1], ridx1, sr1).wait()
        pltpu.sync_copy(buf, acc.at[ridx1], add=True)

        @pl.when(j + 3 < NCHUNK)
        def _pf1():
            pltpu.async_copy(rows_hbm.at[wid, j + 3], ridx1, sr1)

        return carry

    lax.fori_loop(0, NCHUNK // 2, body, 0)
    plsc.subcore_barrier()
    pltpu.sync_copy(acc.at[pl.ds(s * ZROWS_PER_TILE, ZROWS_PER_TILE)],
                    out_hbm.at[c, pl.ds(s * ZROWS_PER_TILE, ZROWS_PER_TILE)])


def _hop_body(g_hbm, rows_hbm, cols_hbm, zeros_hbm, out_hbm,
              cols_v, ridx0, ridx1, buf0, buf1, acc,
              sg0, sg1, sr0, sr1):
    c = lax.axis_index("c")
    s = lax.axis_index("s")
    wid = c * NS + s
    pltpu.sync_copy(zeros_hbm, buf0)
    for z in range(ZROWS_PER_TILE // CHUNK):
        pltpu.sync_copy(buf0, acc.at[pl.ds(s * ZROWS_PER_TILE + z * CHUNK, CHUNK)])
    pltpu.sync_copy(cols_hbm.at[wid], cols_v)
    plsc.subcore_barrier()

    pltpu.async_copy(g_hbm.at[cols_v.at[0]], buf0, sg0)
    pltpu.async_copy(g_hbm.at[cols_v.at[1]], buf1, sg1)
    pltpu.async_copy(rows_hbm.at[wid, 0], ridx0, sr0)
    pltpu.async_copy(rows_hbm.at[wid, 1], ridx1, sr1)

    def body(jj, carry):
        j = jj * 2
        pltpu.make_async_copy(g_hbm.at[cols_v.at[j]], buf0, sg0).wait()
        pltpu.make_async_copy(rows_hbm.at[wid, j], ridx0, sr0).wait()
        pltpu.sync_copy(buf0, acc.at[ridx0], add=True)

        @pl.when(j + 2 < NCHUNK)
        def _pf0():
            pltpu.async_copy(g_hbm.at[cols_v.at[j + 2]], buf0, sg0)
            pltpu.async_copy(rows_hbm.at[wid, j + 2], ridx0, sr0)

        pltpu.make_async_copy(g_hbm.at[cols_v.at[j + 1]], buf1, sg1).wait()
        pltpu.make_async_copy(rows_hbm.at[wid, j + 1], ridx1, sr1).wait()
        pltpu.sync_copy(buf1, acc.at[ridx1], add=True)

        @pl.when(j + 3 < NCHUNK)
        def _pf1():
            pltpu.async_copy(g_hbm.at[cols_v.at[j + 3]], buf1, sg1)
            pltpu.async_copy(rows_hbm.at[wid, j + 3], ridx1, sr1)

        return carry

    lax.fori_loop(0, NCHUNK // 2, body, 0)
    plsc.subcore_barrier()
    pltpu.sync_copy(acc.at[pl.ds(s * ZROWS_PER_TILE, ZROWS_PER_TILE)],
                    out_hbm.at[c, pl.ds(s * ZROWS_PER_TILE, ZROWS_PER_TILE)])


def _run_deg(rows_pad):
    fn = pl.kernel(
        _deg_body,
        out_type=jax.ShapeDtypeStruct((NC, ACC_ROWS, DEG_W), jnp.float32),
        mesh=_sc_mesh(),
        scratch_types=[
            pltpu.VMEM((CHUNK,), jnp.int32),
            pltpu.VMEM((CHUNK,), jnp.int32),
            pltpu.VMEM((CHUNK, DEG_W), jnp.float32),
            pltpu.VMEM_SHARED((ACC_ROWS, DEG_W), jnp.float32),
            pltpu.SemaphoreType.DMA,
            pltpu.SemaphoreType.DMA,
        ],
    )
    zeros = jnp.zeros((CHUNK, DEG_W), jnp.float32)
    ones = jnp.ones((CHUNK, DEG_W), jnp.float32)
    return fn(rows_pad, zeros, ones)


def _run_hop(g, rows_pad, cols_pad):
    fn = pl.kernel(
        _hop_body,
        out_type=jax.ShapeDtypeStruct((NC, ACC_ROWS, D), jnp.float32),
        mesh=_sc_mesh(),
        scratch_types=[
            pltpu.VMEM((NCHUNK, CHUNK), jnp.int32),
            pltpu.VMEM((CHUNK,), jnp.int32),
            pltpu.VMEM((CHUNK,), jnp.int32),
            pltpu.VMEM((CHUNK, D), jnp.float32),
            pltpu.VMEM((CHUNK, D), jnp.float32),
            pltpu.VMEM_SHARED((ACC_ROWS, D), jnp.float32),
            pltpu.SemaphoreType.DMA,
            pltpu.SemaphoreType.DMA,
            pltpu.SemaphoreType.DMA,
            pltpu.SemaphoreType.DMA,
        ],
    )
    zeros = jnp.zeros((CHUNK, D), jnp.float32)
    return fn(g, rows_pad, cols_pad, zeros)


def _mm_body(x_ref, w_ref, y_ref):
    y_ref[...] = lax.dot_general(x_ref[...], w_ref[...],
                                 (((1,), (1,)), ((), ())),
                                 preferred_element_type=jnp.float32)


def _run_mm(x, W):
    grid = N // ROWBLK
    return pl.pallas_call(
        _mm_body,
        grid=(grid,),
        in_specs=[
            pl.BlockSpec((ROWBLK, D), lambda i: (i, 0)),
            pl.BlockSpec((D, D), lambda i: (0, 0)),
        ],
        out_specs=pl.BlockSpec((ROWBLK, D), lambda i: (i, 0)),
        out_shape=jax.ShapeDtypeStruct((N, D), jnp.float32),
    )(x, W)


def _deg_from_parts(dp_ref):
    return dp_ref[0, :, 0:1] + dp_ref[1, :, 0:1] + 1.0


def _scale_body(dp_ref, y_ref, g_ref):
    g_ref[...] = y_ref[...] * lax.rsqrt(_deg_from_parts(dp_ref))


def _run_scale(dp, y):
    grid = N // ROWBLK
    return pl.pallas_call(
        _scale_body,
        grid=(grid,),
        in_specs=[
            pl.BlockSpec((NC, ROWBLK, DEG_W), lambda i: (0, i, 0)),
            pl.BlockSpec((ROWBLK, D), lambda i: (i, 0)),
        ],
        out_specs=pl.BlockSpec((ROWBLK, D), lambda i: (i, 0)),
        out_shape=jax.ShapeDtypeStruct((N, D), jnp.float32),
    )(dp, y)


def _mid_body(dp_ref, p_ref, g0_ref, g1_ref):
    g1_ref[...] = (p_ref[0] + p_ref[1] + g0_ref[...]) / _deg_from_parts(dp_ref)


def _run_mid(dp, p, g0):
    grid = N // ROWBLK
    return pl.pallas_call(
        _mid_body,
        grid=(grid,),
        in_specs=[
            pl.BlockSpec((NC, ROWBLK, DEG_W), lambda i: (0, i, 0)),
            pl.BlockSpec((NC, ROWBLK, D), lambda i: (0, i, 0)),
            pl.BlockSpec((ROWBLK, D), lambda i: (i, 0)),
        ],
        out_specs=pl.BlockSpec((ROWBLK, D), lambda i: (i, 0)),
        out_shape=jax.ShapeDtypeStruct((N, D), jnp.float32),
    )(dp, p, g0)


def _final_body(dp_ref, q_ref, g1_ref, b_ref, o_ref):
    z = ((q_ref[0] + q_ref[1] + g1_ref[...]) * lax.rsqrt(_deg_from_parts(dp_ref))
         + b_ref[...])
    m = jnp.max(z, axis=1, keepdims=True)
    lse = jnp.log(jnp.sum(jnp.exp(z - m), axis=1, keepdims=True)) + m
    o_ref[...] = z - lse


def _run_final(dp, q, g1, b):
    grid = N // ROWBLK
    return pl.pallas_call(
        _final_body,
        grid=(grid,),
        in_specs=[
            pl.BlockSpec((NC, ROWBLK, DEG_W), lambda i: (0, i, 0)),
            pl.BlockSpec((NC, ROWBLK, D), lambda i: (0, i, 0)),
            pl.BlockSpec((ROWBLK, D), lambda i: (i, 0)),
            pl.BlockSpec((1, D), lambda i: (0, 0)),
        ],
        out_specs=pl.BlockSpec((ROWBLK, D), lambda i: (i, 0)),
        out_shape=jax.ShapeDtypeStruct((N, D), jnp.float32),
    )(dp, q, g1, b)


def kernel(x, edge_index, W, b):
    row = edge_index[0].reshape(NW, EPT)
    col = edge_index[1].reshape(NW, EPT)
    pad = EPT_PAD - EPT
    rows_pad = jnp.pad(row, ((0, 0), (0, pad)), constant_values=TRASH)
    cols_pad = jnp.pad(col, ((0, 0), (0, pad)), constant_values=0)
    rows_pad = rows_pad.reshape(NW, NCHUNK, CHUNK)
    cols_pad = cols_pad.reshape(NW, NCHUNK, CHUNK)

    dp = _run_deg(rows_pad)
    y = _run_mm(x, W)
    g0 = _run_scale(dp, y)
    p = _run_hop(g0, rows_pad, cols_pad)
    g1 = _run_mid(dp, p, g0)
    q = _run_hop(g1, rows_pad, cols_pad)
    return _run_final(dp, q, g1, b.reshape(1, D))


# speedup vs baseline: 1.3649x; 1.0013x over previous
import functools

import jax
import jax.numpy as jnp
from jax import lax
from jax.experimental import pallas as pl
from jax.experimental.pallas import tpu as pltpu
from jax.experimental.pallas import tpu_sc as plsc

N = 10000
E = 320000
D = 128
NC = 2
NS = 16
NW = NC * NS
EPT = E // NW
CHUNK = 128
NCHUNK = 80
EPT_PAD = NCHUNK * CHUNK
ACC_ROWS = 10240
TRASH = N + 100
ZROWS_PER_TILE = ACC_ROWS // NS
DEG_W = 128
ROWBLK = 2000


def _sc_mesh():
    return plsc.VectorSubcoreMesh(core_axis_name="c", subcore_axis_name="s")


def _deg_body(rows_hbm, zeros_hbm, ones_hbm, out_hbm,
              ridx0, ridx1, buf, acc, sr0, sr1):
    c = lax.axis_index("c")
    s = lax.axis_index("s")
    wid = c * NS + s
    pltpu.sync_copy(zeros_hbm, buf)
    for z in range(ZROWS_PER_TILE // CHUNK):
        pltpu.sync_copy(buf, acc.at[pl.ds(s * ZROWS_PER_TILE + z * CHUNK, CHUNK)])
    pltpu.sync_copy(ones_hbm, buf)
    plsc.subcore_barrier()
    pltpu.async_copy(rows_hbm.at[wid, 0], ridx0, sr0)
    pltpu.async_copy(rows_hbm.at[wid, 1], ridx1, sr1)

    def body(jj, carry):
        j = jj * 2
        pltpu.make_async_copy(rows_hbm.at[wid, j], ridx0, sr0).wait()
        pltpu.sync_copy(buf, acc.at[ridx0], add=True)

        @pl.when(j + 2 < NCHUNK)
        def _pf0():
            pltpu.async_copy(rows_hbm.at[wid, j + 2], ridx0, sr0)

        pltpu.make_async_copy(rows_hbm.at[wid, j + 1], ridx1, sr1).wait()
        pltpu.sync_copy(buf, acc.at[ridx1], add=True)

        @pl.when(j + 3 < NCHUNK)
        def _pf1():
            pltpu.async_copy(rows_hbm.at[wid, j + 3], ridx1, sr1)

        return carry

    lax.fori_loop(0, NCHUNK // 2, body, 0)
    plsc.subcore_barrier()
    pltpu.sync_copy(acc.at[pl.ds(s * ZROWS_PER_TILE, ZROWS_PER_TILE)],
                    out_hbm.at[c, pl.ds(s * ZROWS_PER_TILE, ZROWS_PER_TILE)])


def _hop_body(g_hbm, rows_hbm, cols_hbm, zeros_hbm, out_hbm,
              cols_v, ridx0, ridx1, buf0, buf1, acc,
              sg0, sg1, sr0, sr1):
    c = lax.axis_index("c")
    s = lax.axis_index("s")
    wid = c * NS + s
    pltpu.sync_copy(zeros_hbm, buf0)
    for z in range(ZROWS_PER_TILE // CHUNK):
        pltpu.sync_copy(buf0, acc.at[pl.ds(s * ZROWS_PER_TILE + z * CHUNK, CHUNK)])
    pltpu.sync_copy(cols_hbm.at[wid], cols_v)
    plsc.subcore_barrier()

    pltpu.async_copy(g_hbm.at[cols_v.at[0]], buf0, sg0)
    pltpu.async_copy(g_hbm.at[cols_v.at[1]], buf1, sg1)
    pltpu.async_copy(rows_hbm.at[wid, 0], ridx0, sr0)
    pltpu.async_copy(rows_hbm.at[wid, 1], ridx1, sr1)

    def body(jj, carry):
        j = jj * 2
        pltpu.make_async_copy(g_hbm.at[cols_v.at[j]], buf0, sg0).wait()
        pltpu.make_async_copy(rows_hbm.at[wid, j], ridx0, sr0).wait()
        pltpu.sync_copy(buf0, acc.at[ridx0], add=True)

        @pl.when(j + 2 < NCHUNK)
        def _pf0():
            pltpu.async_copy(g_hbm.at[cols_v.at[j + 2]], buf0, sg0)
            pltpu.async_copy(rows_hbm.at[wid, j + 2], ridx0, sr0)

        pltpu.make_async_copy(g_hbm.at[cols_v.at[j + 1]], buf1, sg1).wait()
        pltpu.make_async_copy(rows_hbm.at[wid, j + 1], ridx1, sr1).wait()
        pltpu.sync_copy(buf1, acc.at[ridx1], add=True)

        @pl.when(j + 3 < NCHUNK)
        def _pf1():
            pltpu.async_copy(g_hbm.at[cols_v.at[j + 3]], buf1, sg1)
            pltpu.async_copy(rows_hbm.at[wid, j + 3], ridx1, sr1)

        return carry

    lax.fori_loop(0, NCHUNK // 2, body, 0)
    plsc.subcore_barrier()
    pltpu.sync_copy(acc.at[pl.ds(s * ZROWS_PER_TILE, ZROWS_PER_TILE)],
                    out_hbm.at[c, pl.ds(s * ZROWS_PER_TILE, ZROWS_PER_TILE)])


def _run_deg(rows_pad):
    fn = pl.kernel(
        _deg_body,
        out_type=jax.ShapeDtypeStruct((NC, ACC_ROWS, DEG_W), jnp.float32),
        mesh=_sc_mesh(),
        scratch_types=[
            pltpu.VMEM((CHUNK,), jnp.int32),
            pltpu.VMEM((CHUNK,), jnp.int32),
            pltpu.VMEM((CHUNK, DEG_W), jnp.float32),
            pltpu.VMEM_SHARED((ACC_ROWS, DEG_W), jnp.float32),
            pltpu.SemaphoreType.DMA,
            pltpu.SemaphoreType.DMA,
        ],
    )
    zeros = jnp.zeros((CHUNK, DEG_W), jnp.float32)
    ones = jnp.ones((CHUNK, DEG_W), jnp.float32)
    return fn(rows_pad, zeros, ones)


def _run_hop(g, rows_pad, cols_pad):
    fn = pl.kernel(
        _hop_body,
        out_type=jax.ShapeDtypeStruct((NC, ACC_ROWS, D), jnp.float32),
        mesh=_sc_mesh(),
        scratch_types=[
            pltpu.VMEM((NCHUNK, CHUNK), jnp.int32),
            pltpu.VMEM((CHUNK,), jnp.int32),
            pltpu.VMEM((CHUNK,), jnp.int32),
            pltpu.VMEM((CHUNK, D), jnp.float32),
            pltpu.VMEM((CHUNK, D), jnp.float32),
            pltpu.VMEM_SHARED((ACC_ROWS, D), jnp.float32),
            pltpu.SemaphoreType.DMA,
            pltpu.SemaphoreType.DMA,
            pltpu.SemaphoreType.DMA,
            pltpu.SemaphoreType.DMA,
        ],
    )
    zeros = jnp.zeros((CHUNK, D), jnp.float32)
    return fn(g, rows_pad, cols_pad, zeros)


def _mm_body(dp_ref, x_ref, w_ref, y_ref):
    y = lax.dot_general(x_ref[...], w_ref[...],
                        (((1,), (1,)), ((), ())),
                        preferred_element_type=jnp.float32)
    y_ref[...] = y * lax.rsqrt(_deg_from_parts(dp_ref))


def _run_mm_scale(dp, x, W):
    grid = N // ROWBLK
    return pl.pallas_call(
        _mm_body,
        grid=(grid,),
        in_specs=[
            pl.BlockSpec((NC, ROWBLK, DEG_W), lambda i: (0, i, 0)),
            pl.BlockSpec((ROWBLK, D), lambda i: (i, 0)),
            pl.BlockSpec((D, D), lambda i: (0, 0)),
        ],
        out_specs=pl.BlockSpec((ROWBLK, D), lambda i: (i, 0)),
        out_shape=jax.ShapeDtypeStruct((N, D), jnp.float32),
    )(dp, x, W)


def _deg_from_parts(dp_ref):
    return dp_ref[0, :, 0:1] + dp_ref[1, :, 0:1] + 1.0


def _scale_body(dp_ref, y_ref, g_ref):
    g_ref[...] = y_ref[...] * lax.rsqrt(_deg_from_parts(dp_ref))


def _run_scale(dp, y):
    grid = N // ROWBLK
    return pl.pallas_call(
        _scale_body,
        grid=(grid,),
        in_specs=[
            pl.BlockSpec((NC, ROWBLK, DEG_W), lambda i: (0, i, 0)),
            pl.BlockSpec((ROWBLK, D), lambda i: (i, 0)),
        ],
        out_specs=pl.BlockSpec((ROWBLK, D), lambda i: (i, 0)),
        out_shape=jax.ShapeDtypeStruct((N, D), jnp.float32),
    )(dp, y)


def _mid_body(dp_ref, p_ref, g0_ref, g1_ref):
    g1_ref[...] = (p_ref[0] + p_ref[1] + g0_ref[...]) / _deg_from_parts(dp_ref)


def _run_mid(dp, p, g0):
    grid = N // ROWBLK
    return pl.pallas_call(
        _mid_body,
        grid=(grid,),
        in_specs=[
            pl.BlockSpec((NC, ROWBLK, DEG_W), lambda i: (0, i, 0)),
            pl.BlockSpec((NC, ROWBLK, D), lambda i: (0, i, 0)),
            pl.BlockSpec((ROWBLK, D), lambda i: (i, 0)),
        ],
        out_specs=pl.BlockSpec((ROWBLK, D), lambda i: (i, 0)),
        out_shape=jax.ShapeDtypeStruct((N, D), jnp.float32),
    )(dp, p, g0)


def _final_body(dp_ref, q_ref, g1_ref, b_ref, o_ref):
    z = ((q_ref[0] + q_ref[1] + g1_ref[...]) * lax.rsqrt(_deg_from_parts(dp_ref))
         + b_ref[...])
    m = jnp.max(z, axis=1, keepdims=True)
    lse = jnp.log(jnp.sum(jnp.exp(z - m), axis=1, keepdims=True)) + m
    o_ref[...] = z - lse


def _run_final(dp, q, g1, b):
    grid = N // ROWBLK
    return pl.pallas_call(
        _final_body,
        grid=(grid,),
        in_specs=[
            pl.BlockSpec((NC, ROWBLK, DEG_W), lambda i: (0, i, 0)),
            pl.BlockSpec((NC, ROWBLK, D), lambda i: (0, i, 0)),
            pl.BlockSpec((ROWBLK, D), lambda i: (i, 0)),
            pl.BlockSpec((1, D), lambda i: (0, 0)),
        ],
        out_specs=pl.BlockSpec((ROWBLK, D), lambda i: (i, 0)),
        out_shape=jax.ShapeDtypeStruct((N, D), jnp.float32),
    )(dp, q, g1, b)


def kernel(x, edge_index, W, b):
    row = edge_index[0].reshape(NW, EPT)
    col = edge_index[1].reshape(NW, EPT)
    pad = EPT_PAD - EPT
    rows_pad = jnp.pad(row, ((0, 0), (0, pad)), constant_values=TRASH)
    cols_pad = jnp.pad(col, ((0, 0), (0, pad)), constant_values=0)
    rows_pad = rows_pad.reshape(NW, NCHUNK, CHUNK)
    cols_pad = cols_pad.reshape(NW, NCHUNK, CHUNK)

    dp = _run_deg(rows_pad)
    g0 = _run_mm_scale(dp, x, W)
    p = _run_hop(g0, rows_pad, cols_pad)
    g1 = _run_mid(dp, p, g0)
    q = _run_hop(g1, rows_pad, cols_pad)
    return _run_final(dp, q, g1, b.reshape(1, D))
